# baseline (speedup 1.0000x reference)
import functools

import jax
import jax.numpy as jnp
from jax import lax
from jax.experimental import pallas as pl
from jax.experimental.pallas import tpu as pltpu
from jax.experimental.pallas import tpu_sc as plsc

NUM_ROWS = 1000000
DIM = 32
NUM_CORES = 2
NUM_SUBCORES = 16
NUM_WORKERS = NUM_CORES * NUM_SUBCORES
ROW_GROUPS = 4
GROUP_ROWS = DIM // ROW_GROUPS
COL_RANGES = NUM_WORKERS // ROW_GROUPS
COLS_PER_RANGE = 124928
CHUNK = 7808
HALF_A = 3968
HALF_B = CHUNK - HALF_A
PAIRS = COLS_PER_RANGE // (2 * CHUNK)
TAIL_BASE = COL_RANGES * COLS_PER_RANGE
TAIL = NUM_ROWS - TAIL_BASE


@functools.partial(
    pl.kernel,
    out_type=jax.ShapeDtypeStruct((DIM, NUM_ROWS), jnp.float32),
    mesh=plsc.VectorSubcoreMesh(core_axis_name="c", subcore_axis_name="s"),
    scratch_types=[
        pltpu.VMEM((GROUP_ROWS, CHUNK), jnp.float32),
        pltpu.VMEM((GROUP_ROWS, CHUNK), jnp.float32),
        pltpu.VMEM((GROUP_ROWS, TAIL), jnp.float32),
        pltpu.SemaphoreType.DMA,
        pltpu.SemaphoreType.DMA,
        pltpu.SemaphoreType.DMA,
        pltpu.SemaphoreType.DMA,
    ],
)
def _copy_table_t(emb, out, buf0, buf1, tailbuf, si0, si1, so0, so1):
    wid = lax.axis_index("s") * NUM_CORES + lax.axis_index("c")
    row0 = (wid // COL_RANGES) * GROUP_ROWS
    col0 = (wid % COL_RANGES) * COLS_PER_RANGE

    def src(i):
        return emb.at[pl.ds(row0, GROUP_ROWS), pl.ds(col0 + i * CHUNK, CHUNK)]

    def dst(i):
        return out.at[pl.ds(row0, GROUP_ROWS), pl.ds(col0 + i * CHUNK, CHUNK)]

    def body(g, carry):
        i0 = 2 * g
        i1 = i0 + 1

        @pl.when(g > 0)
        def _():
            pltpu.make_async_copy(buf0, dst(i0), so0).wait()
            pltpu.make_async_copy(buf1, dst(i1), so1).wait()

        def halves(ref2d):
            return (ref2d.at[:, pl.ds(0, HALF_A)],
                    ref2d.at[:, pl.ds(HALF_A, HALF_B)])

        sa0, sb0 = halves(src(i0))
        sa1, sb1 = halves(src(i1))
        da0, db0 = halves(dst(i0))
        da1, db1 = halves(dst(i1))
        ba0, bb0 = halves(buf0)
        ba1, bb1 = halves(buf1)
        pltpu.async_copy(sa0, ba0, si0)
        pltpu.async_copy(sb0, bb0, si0)
        pltpu.async_copy(sa1, ba1, si1)
        pltpu.async_copy(sb1, bb1, si1)
        pltpu.make_async_copy(sa0, ba0, si0).wait()
        pltpu.async_copy(ba0, da0, so0)
        pltpu.make_async_copy(sb0, bb0, si0).wait()
        pltpu.async_copy(bb0, db0, so0)
        pltpu.make_async_copy(sa1, ba1, si1).wait()
        pltpu.async_copy(ba1, da1, so1)
        pltpu.make_async_copy(sb1, bb1, si1).wait()
        pltpu.async_copy(bb1, db1, so1)
        return carry

    lax.fori_loop(0, PAIRS, body, 0)
    pltpu.make_async_copy(buf0, dst(0), so0).wait()
    pltpu.make_async_copy(buf1, dst(1), so1).wait()

    @pl.when(wid % COL_RANGES == COL_RANGES - 1)
    def _():
        pltpu.sync_copy(
            emb.at[pl.ds(row0, GROUP_ROWS), pl.ds(TAIL_BASE, TAIL)],
            tailbuf,
        )
        pltpu.sync_copy(
            tailbuf,
            out.at[pl.ds(row0, GROUP_ROWS), pl.ds(TAIL_BASE, TAIL)],
        )


def kernel(inp, embedding):
    del inp
    out_t = _copy_table_t(jnp.swapaxes(embedding, 0, 1))
    return jnp.swapaxes(out_t, 0, 1)
